# baseline (speedup 1.0000x reference)
import functools

import jax
import jax.numpy as jnp
from jax import lax
from jax.experimental import pallas as pl
from jax.experimental.pallas import tpu as pltpu
from jax.experimental.pallas import tpu_sc as plsc

_VOCAB = 1000
_BATCH = 16384
_D = 1000
_VPAD = 1024
_NC = 2
_NS = 16
_NW = _NC * _NS
_FB = 32
_LAST_START = _D - _FB
_LANES = 16
_BCHUNK = 256
_NBCHUNK = _BATCH // _BCHUNK
_GROUPS = _BCHUNK // _LANES


def _body(t1t_hbm, t2t_hbm, idx0_hbm, idx1_hbm, out_hbm,
          p1, p2, idx0_v, idx1_v, st0, st1, sem_w0, sem_w1, sem_p):
    wid = lax.axis_index("s") * _NC + lax.axis_index("c")
    fstart = jnp.minimum(wid * _FB, _LAST_START)
    for r in range(_FB):
        pltpu.async_copy(t1t_hbm.at[fstart + r],
                         p1.at[pl.ds(r * _VPAD, _VPAD)], sem_p)
        pltpu.async_copy(t2t_hbm.at[fstart + r],
                         p2.at[pl.ds(r * _VPAD, _VPAD)], sem_p)
    pltpu.sync_copy(idx0_hbm, idx0_v)
    pltpu.sync_copy(idx1_hbm, idx1_v)
    for r in range(_FB):
        pltpu.make_async_copy(t1t_hbm.at[fstart + r],
                              p1.at[pl.ds(r * _VPAD, _VPAD)], sem_p).wait()
        pltpu.make_async_copy(t2t_hbm.at[fstart + r],
                              p2.at[pl.ds(r * _VPAD, _VPAD)], sem_p).wait()

    def compute_chunk(c, stage):
        b0 = c * _BCHUNK

        @plsc.parallel_loop(0, _GROUPS, 1, unroll=2)
        def group_body(g):
            off = g * _LANES
            v0 = idx0_v[pl.ds(b0 + off, _LANES)]
            v1 = idx1_v[pl.ds(b0 + off, _LANES)]
            step = jnp.int32(_VPAD)
            ga0 = plsc.load_gather(p1, [v0])
            gb0 = plsc.load_gather(p2, [v1])
            v0 = v0 + step
            v1 = v1 + step
            ga1 = plsc.load_gather(p1, [v0])
            gb1 = plsc.load_gather(p2, [v1])
            v0 = v0 + step
            v1 = v1 + step
            ga2 = plsc.load_gather(p1, [v0])
            gb2 = plsc.load_gather(p2, [v1])
            for f in range(3, _FB):
                v0 = v0 + step
                v1 = v1 + step
                ga3 = plsc.load_gather(p1, [v0])
                gb3 = plsc.load_gather(p2, [v1])
                stage[f - 3, pl.ds(off, _LANES)] = ga0 + gb0
                ga0, gb0 = ga1, gb1
                ga1, gb1 = ga2, gb2
                ga2, gb2 = ga3, gb3
            stage[_FB - 3, pl.ds(off, _LANES)] = ga0 + gb0
            stage[_FB - 2, pl.ds(off, _LANES)] = ga1 + gb1
            stage[_FB - 1, pl.ds(off, _LANES)] = ga2 + gb2

    def out_slice(c):
        return out_hbm.at[pl.ds(fstart, _FB), pl.ds(c * _BCHUNK, _BCHUNK)]

    def pair_body(k, carry):
        c0 = 2 * k
        c1 = 2 * k + 1

        @pl.when(k > 0)
        def _():
            pltpu.make_async_copy(st0, out_slice(c0), sem_w0).wait()
        compute_chunk(c0, st0)
        pltpu.async_copy(st0, out_slice(c0), sem_w0)

        @pl.when(k > 0)
        def _():
            pltpu.make_async_copy(st1, out_slice(c1), sem_w1).wait()
        compute_chunk(c1, st1)
        pltpu.async_copy(st1, out_slice(c1), sem_w1)
        return carry

    lax.fori_loop(0, _NBCHUNK // 2, pair_body, 0)
    pltpu.make_async_copy(st0, out_slice(_NBCHUNK - 2), sem_w0).wait()
    pltpu.make_async_copy(st1, out_slice(_NBCHUNK - 1), sem_w1).wait()


@jax.jit
def _sc_bigram_t(t1t, t2t, idx0, idx1):
    mesh = plsc.VectorSubcoreMesh(core_axis_name="c", subcore_axis_name="s")
    f = functools.partial(
        pl.kernel,
        out_type=jax.ShapeDtypeStruct((_D, _BATCH), jnp.float32),
        mesh=mesh,
        scratch_types=[
            pltpu.VMEM((_FB * _VPAD,), jnp.float32),
            pltpu.VMEM((_FB * _VPAD,), jnp.float32),
            pltpu.VMEM((_BATCH,), jnp.int32),
            pltpu.VMEM((_BATCH,), jnp.int32),
            pltpu.VMEM((_FB, _BCHUNK), jnp.float32),
            pltpu.VMEM((_FB, _BCHUNK), jnp.float32),
            pltpu.SemaphoreType.DMA,
            pltpu.SemaphoreType.DMA,
            pltpu.SemaphoreType.DMA,
        ],
        compiler_params=pltpu.CompilerParams(needs_layout_passes=False),
    )(_body)
    return f(t1t, t2t, idx0, idx1)


def kernel(idx, table1, table2):
    if idx.ndim == 1:
        idx = idx[:, None]
    idx = idx.astype(jnp.int32)
    idx0 = idx[:, 0]
    idx1 = idx[:, 1]
    pad = ((0, 0), (0, _VPAD - _VOCAB))
    t1t = jnp.pad(table1.T, pad)
    t2t = jnp.pad(table2.T, pad)
    out_t = _sc_bigram_t(t1t, t2t, idx0, idx1)
    return out_t.T


# speedup vs baseline: 1.2151x; 1.2151x over previous
import functools

import jax
import jax.numpy as jnp
from jax import lax
from jax.experimental import pallas as pl
from jax.experimental.pallas import tpu as pltpu
from jax.experimental.pallas import tpu_sc as plsc

_VOCAB = 1000
_BATCH = 16384
_D = 1000
_VPAD = 1024
_NC = 2
_NS = 16
_NW = _NC * _NS
_FB = 32
_LAST_START = _D - _FB
_LANES = 16
_BCHUNK = 256
_NBCHUNK = _BATCH // _BCHUNK
_GROUPS = _BCHUNK // _LANES


def _body(t1t_hbm, t2t_hbm, idx0_hbm, idx1_hbm, out_hbm,
          p1, p2, idx0_v, idx1_v, st0, st1, sem_w0, sem_w1, sem_p):
    wid = lax.axis_index("s") * _NC + lax.axis_index("c")
    fstart = jnp.minimum(wid * _FB, _LAST_START)
    for r in range(_FB):
        pltpu.async_copy(t1t_hbm.at[fstart + r],
                         p1.at[pl.ds(r * _VPAD, _VPAD)], sem_p)
        pltpu.async_copy(t2t_hbm.at[fstart + r],
                         p2.at[pl.ds(r * _VPAD, _VPAD)], sem_p)
    pltpu.sync_copy(idx0_hbm, idx0_v)
    pltpu.sync_copy(idx1_hbm, idx1_v)
    for r in range(_FB):
        pltpu.make_async_copy(t1t_hbm.at[fstart + r],
                              p1.at[pl.ds(r * _VPAD, _VPAD)], sem_p).wait()
        pltpu.make_async_copy(t2t_hbm.at[fstart + r],
                              p2.at[pl.ds(r * _VPAD, _VPAD)], sem_p).wait()

    def compute_chunk(c, stage):
        b0 = c * _BCHUNK

        def group_pair_body(h, carry):
            off_a = (2 * h) * _LANES
            off_b = off_a + _LANES
            step = jnp.int32(_VPAD)
            va0 = idx0_v[pl.ds(b0 + off_a, _LANES)]
            va1 = idx1_v[pl.ds(b0 + off_a, _LANES)]
            vb0 = idx0_v[pl.ds(b0 + off_b, _LANES)]
            vb1 = idx1_v[pl.ds(b0 + off_b, _LANES)]
            pa0 = plsc.load_gather(p1, [va0])
            qa0 = plsc.load_gather(p2, [va1])
            pb0 = plsc.load_gather(p1, [vb0])
            qb0 = plsc.load_gather(p2, [vb1])
            va0 = va0 + step
            va1 = va1 + step
            vb0 = vb0 + step
            vb1 = vb1 + step
            pa1 = plsc.load_gather(p1, [va0])
            qa1 = plsc.load_gather(p2, [va1])
            pb1 = plsc.load_gather(p1, [vb0])
            qb1 = plsc.load_gather(p2, [vb1])
            for f in range(2, _FB):
                va0 = va0 + step
                va1 = va1 + step
                vb0 = vb0 + step
                vb1 = vb1 + step
                pa2 = plsc.load_gather(p1, [va0])
                qa2 = plsc.load_gather(p2, [va1])
                pb2 = plsc.load_gather(p1, [vb0])
                qb2 = plsc.load_gather(p2, [vb1])
                stage[f - 2, pl.ds(off_a, _LANES)] = pa0 + qa0
                stage[f - 2, pl.ds(off_b, _LANES)] = pb0 + qb0
                pa0, qa0, pb0, qb0 = pa1, qa1, pb1, qb1
                pa1, qa1, pb1, qb1 = pa2, qa2, pb2, qb2
            stage[_FB - 2, pl.ds(off_a, _LANES)] = pa0 + qa0
            stage[_FB - 2, pl.ds(off_b, _LANES)] = pb0 + qb0
            stage[_FB - 1, pl.ds(off_a, _LANES)] = pa1 + qa1
            stage[_FB - 1, pl.ds(off_b, _LANES)] = pb1 + qb1
            return carry

        lax.fori_loop(0, _GROUPS // 2, group_pair_body, 0)

    def out_slice(c):
        return out_hbm.at[pl.ds(fstart, _FB), pl.ds(c * _BCHUNK, _BCHUNK)]

    def pair_body(k, carry):
        c0 = 2 * k
        c1 = 2 * k + 1

        @pl.when(k > 0)
        def _():
            pltpu.make_async_copy(st0, out_slice(c0), sem_w0).wait()
        compute_chunk(c0, st0)
        pltpu.async_copy(st0, out_slice(c0), sem_w0)

        @pl.when(k > 0)
        def _():
            pltpu.make_async_copy(st1, out_slice(c1), sem_w1).wait()
        compute_chunk(c1, st1)
        pltpu.async_copy(st1, out_slice(c1), sem_w1)
        return carry

    lax.fori_loop(0, _NBCHUNK // 2, pair_body, 0)
    pltpu.make_async_copy(st0, out_slice(_NBCHUNK - 2), sem_w0).wait()
    pltpu.make_async_copy(st1, out_slice(_NBCHUNK - 1), sem_w1).wait()


@jax.jit
def _sc_bigram_t(t1t, t2t, idx0, idx1):
    mesh = plsc.VectorSubcoreMesh(core_axis_name="c", subcore_axis_name="s")
    f = functools.partial(
        pl.kernel,
        out_type=jax.ShapeDtypeStruct((_D, _BATCH), jnp.float32),
        mesh=mesh,
        scratch_types=[
            pltpu.VMEM((_FB * _VPAD,), jnp.float32),
            pltpu.VMEM((_FB * _VPAD,), jnp.float32),
            pltpu.VMEM((_BATCH,), jnp.int32),
            pltpu.VMEM((_BATCH,), jnp.int32),
            pltpu.VMEM((_FB, _BCHUNK), jnp.float32),
            pltpu.VMEM((_FB, _BCHUNK), jnp.float32),
            pltpu.SemaphoreType.DMA,
            pltpu.SemaphoreType.DMA,
            pltpu.SemaphoreType.DMA,
        ],
        compiler_params=pltpu.CompilerParams(needs_layout_passes=False),
    )(_body)
    return f(t1t, t2t, idx0, idx1)


def kernel(idx, table1, table2):
    if idx.ndim == 1:
        idx = idx[:, None]
    idx = idx.astype(jnp.int32)
    idx0 = idx[:, 0]
    idx1 = idx[:, 1]
    pad = ((0, 0), (0, _VPAD - _VOCAB))
    t1t = jnp.pad(table1.T, pad)
    t2t = jnp.pad(table2.T, pad)
    out_t = _sc_bigram_t(t1t, t2t, idx0, idx1)
    return out_t.T
